# Initial kernel scaffold; baseline (speedup 1.0000x reference)
import jax
import jax.numpy as jnp
from jax.experimental import pallas as pl


def kernel(x, bin_tokens, Wq, Wk, Wv):
    raise NotImplementedError("write your pallas kernel here")


# speedup vs baseline: 2.7686x; 2.7686x over previous
import jax
import jax.numpy as jnp
from jax import lax
from jax.experimental import pallas as pl
from jax.experimental.pallas import tpu as pltpu
from jax.experimental.pallas import tpu_sc as plsc

_B, _C, _N, _NB, _M = 4, 64, 4096, 4, 1024
_TQ = 512
_NQT = _N // _TQ
_TI = 256
_NIT = _N // _TI
_W = _N + 128
_PREC = None


def _stats_kernel(x_full_ref, x_tile_ref, bins_ref, wq_ref, wk_ref,
                  aps8_ref, bp8_ref, xt_ref, k_scr):
    t = pl.program_id(1)

    @pl.when(t == 0)
    def _():
        wk = wk_ref[...]
        k_main = lax.dot_general(wk, x_full_ref[0], (((1,), (0,)), ((), ())),
                                 preferred_element_type=jnp.float32,
                                 precision=_PREC)
        bins = bins_ref[0]
        binp = jnp.concatenate(
            [bins, jnp.zeros((_C, 128 - _NB), jnp.float32)], axis=1)
        k_bin = lax.dot_general(wk, binp, (((1,), (0,)), ((), ())),
                                preferred_element_type=jnp.float32,
                                precision=_PREC)
        k_scr[:, :_N] = k_main
        k_scr[:, _N:] = k_bin
        aps8_ref[...] = jnp.zeros_like(aps8_ref)
        bp8_ref[...] = jnp.zeros_like(bp8_ref)

    x_tile = x_tile_ref[0]
    q = lax.dot_general(wq_ref[...], x_tile, (((1,), (0,)), ((), ())),
                        preferred_element_type=jnp.float32, precision=_PREC)
    e = lax.dot_general(q, k_scr[...], (((0,), (0,)), ((), ())),
                        preferred_element_type=jnp.float32, precision=_PREC)
    col = lax.broadcasted_iota(jnp.int32, (_TQ, _W), 1)
    e = jnp.where(col < _N + _NB, e, -jnp.inf)
    m = jnp.max(e, axis=1, keepdims=True)
    p = jnp.exp(e - m)
    zacc = p[:, 0:128]
    for i in range(1, _W // 128):
        zacc = zacc + p[:, 128 * i:128 * (i + 1)]
    z = jnp.sum(zacc, axis=1, keepdims=True)
    attn = p / z
    acc = aps8_ref[0]
    bacc = bp8_ref[0]
    for g in range(_TQ // 8):
        rows = attn[8 * g:8 * (g + 1), :]
        acc = acc + rows[:, :_N]
        bacc = jnp.maximum(bacc, rows[:, _N:])
    aps8_ref[0] = acc
    bp8_ref[0] = bacc
    xt_ref[0] = jnp.concatenate(
        [lax.transpose(x_tile, (1, 0)), jnp.zeros((_TQ, 128 - _C), jnp.float32)],
        axis=1)


def _rank_kernel(aps8_ref, bp8_ref, rank_ref, cumks_ref, aps_s, apst_s):
    it = pl.program_id(1)

    @pl.when(it == 0)
    def _():
        a8 = aps8_ref[0]
        t1 = a8[0:4] + a8[4:8]
        t2 = t1[0:2] + t1[2:4]
        aps_s[...] = t2[0:1] + t2[1:2]
        a8t = lax.transpose(a8, (1, 0))
        u1 = a8t[:, 0:4] + a8t[:, 4:8]
        u2 = u1[:, 0:2] + u1[:, 2:4]
        apst_s[...] = u2[:, 0:1] + u2[:, 1:2]
        b8 = bp8_ref[0]
        m1 = jnp.maximum(b8[0:4], b8[4:8])
        m2 = jnp.maximum(m1[0:2], m1[2:4])
        bp = jnp.maximum(m2[0:1], m2[1:2])
        kki = jnp.floor(512.0 * bp).astype(jnp.int32)
        lane = lax.broadcasted_iota(jnp.int32, (1, 128), 1)
        mtot = jnp.full((1, 1), _M, jnp.int32)
        s = jnp.zeros((1, 1), jnp.int32)
        ck = jnp.zeros((1, 128), jnp.int32)
        for j in range(_NB - 1):
            kkj = jnp.sum(jnp.where(lane == j, kki, 0), axis=1, keepdims=True)
            kj = jnp.minimum(kkj, mtot - s)
            ck = ck + jnp.where(lane == 8 + j, kj, 0)
            s = s + kj
            ck = ck + jnp.where(lane == j + 1, s, 0)
        ck = ck + jnp.where(lane == 8 + _NB - 1, mtot - s, 0)
        ck = ck + jnp.where(lane == _NB, mtot, 0)
        cumks_ref[0] = ck

    i0 = it * _TI
    ai = apst_s[pl.ds(i0, _TI), :]
    arow = aps_s[...]
    gt = arow > ai
    eq = arow == ai
    jidx = lax.broadcasted_iota(jnp.int32, (_TI, _N), 1)
    iidx = lax.broadcasted_iota(jnp.int32, (_TI, 1), 0) + i0
    cond = gt | (eq & (jidx < iidx))
    rank_ref[0] = jnp.sum(cond.astype(jnp.int32), axis=1, keepdims=True)


def _stats_call(x, bin_tokens, wq, wk):
    return pl.pallas_call(
        _stats_kernel,
        grid=(_B, _NQT),
        in_specs=[
            pl.BlockSpec((1, _C, _N), lambda b, t: (b, 0, 0)),
            pl.BlockSpec((1, _C, _TQ), lambda b, t: (b, 0, t)),
            pl.BlockSpec((1, _C, _NB), lambda b, t: (0, 0, 0)),
            pl.BlockSpec((_C, _C), lambda b, t: (0, 0)),
            pl.BlockSpec((_C, _C), lambda b, t: (0, 0)),
        ],
        out_specs=[
            pl.BlockSpec((1, 8, _N), lambda b, t: (b, 0, 0)),
            pl.BlockSpec((1, 8, 128), lambda b, t: (b, 0, 0)),
            pl.BlockSpec((1, _TQ, 128), lambda b, t: (b, t, 0)),
        ],
        out_shape=[
            jax.ShapeDtypeStruct((_B, 8, _N), jnp.float32),
            jax.ShapeDtypeStruct((_B, 8, 128), jnp.float32),
            jax.ShapeDtypeStruct((_B, _N, 128), jnp.float32),
        ],
        scratch_shapes=[pltpu.VMEM((_C, _W), jnp.float32)],
        compiler_params=pltpu.CompilerParams(
            dimension_semantics=("arbitrary", "arbitrary")),
    )(x, x, bin_tokens, wq, wk)


def _rank_call(aps8, bp8):
    return pl.pallas_call(
        _rank_kernel,
        grid=(_B, _NIT),
        in_specs=[
            pl.BlockSpec((1, 8, _N), lambda b, it: (b, 0, 0)),
            pl.BlockSpec((1, 8, 128), lambda b, it: (b, 0, 0)),
        ],
        out_specs=[
            pl.BlockSpec((1, _TI, 1), lambda b, it: (b, it, 0)),
            pl.BlockSpec((1, 1, 128), lambda b, it: (b, 0, 0)),
        ],
        out_shape=[
            jax.ShapeDtypeStruct((_B, _N, 1), jnp.int32),
            jax.ShapeDtypeStruct((_B, 1, 128), jnp.int32),
        ],
        scratch_shapes=[
            pltpu.VMEM((1, _N), jnp.float32),
            pltpu.VMEM((_N, 1), jnp.float32),
        ],
        compiler_params=pltpu.CompilerParams(
            dimension_semantics=("arbitrary", "arbitrary")),
    )(aps8, bp8)


def _select_call(rank, cumks):
    mesh = plsc.VectorSubcoreMesh(core_axis_name="c", subcore_axis_name="s")

    @pl.kernel(
        out_type=jax.ShapeDtypeStruct((_B, _M), jnp.int32),
        mesh=mesh,
        scratch_types=[
            pltpu.VMEM((_N,), jnp.int32),
            pltpu.VMEM((128,), jnp.int32),
            pltpu.VMEM((_M,), jnp.int32),
        ],
        compiler_params=pltpu.CompilerParams(needs_layout_passes=False),
    )
    def sel_kernel(rank_hbm, ck_hbm, idx_hbm, rank_v, ck_v, idx_v):
        cid = lax.axis_index("c")
        sid = lax.axis_index("s")

        @pl.when((cid == 0) & (sid < _B))
        def _():
            b = sid
            pltpu.sync_copy(rank_hbm.at[b], rank_v)
            pltpu.sync_copy(ck_hbm.at[b], ck_v)

            @pl.loop(0, _N // 16)
            def _(i):
                r = rank_v[pl.ds(i * 16, 16)]
                j = jnp.right_shift(r, 10)
                t = jnp.bitwise_and(r, 1023)
                cumj = plsc.load_gather(ck_v, [j])
                ksj = plsc.load_gather(ck_v, [j + 8])
                p = cumj + t
                iv = lax.iota(jnp.int32, 16) + i * 16
                plsc.store_scatter(idx_v, [p], iv, mask=t < ksj)

            pltpu.sync_copy(idx_v, idx_hbm.at[b])

    return sel_kernel(rank, cumks)


def _gather_call(idx, xt):
    mesh = plsc.VectorSubcoreMesh(core_axis_name="c", subcore_axis_name="s")
    chunk = _M // 8

    @pl.kernel(
        out_type=jax.ShapeDtypeStruct((_B, _M, 128), jnp.float32),
        mesh=mesh,
        scratch_types=[
            pltpu.VMEM((chunk,), jnp.int32),
            pltpu.VMEM((chunk, 128), jnp.float32),
            pltpu.SemaphoreType.DMA,
        ],
        compiler_params=pltpu.CompilerParams(needs_layout_passes=False),
    )
    def gather_kernel(idx_hbm, xt_hbm, xg_hbm, idx_v, rows_v, sem):
        cid = lax.axis_index("c")
        sid = lax.axis_index("s")
        w = sid * 2 + cid
        b = jnp.right_shift(w, 3)
        ch = jnp.bitwise_and(w, 7)
        pltpu.sync_copy(idx_hbm.at[b].at[pl.ds(ch * chunk, chunk)], idx_v)
        pltpu.async_copy(xt_hbm.at[b].at[idx_v], rows_v, sem).wait()
        pltpu.sync_copy(rows_v, xg_hbm.at[b].at[pl.ds(ch * chunk, chunk)])

    return gather_kernel(idx, xt)


def _v_kernel(wv_ref, xg_ref, o_ref):
    xg = xg_ref[0][:, :_C]
    o_ref[0] = lax.dot_general(wv_ref[...], xg, (((1,), (1,)), ((), ())),
                               preferred_element_type=jnp.float32,
                               precision=_PREC)


def _v_call(wv, xg):
    return pl.pallas_call(
        _v_kernel,
        grid=(_B,),
        in_specs=[
            pl.BlockSpec((_C, _C), lambda b: (0, 0)),
            pl.BlockSpec((1, _M, 128), lambda b: (b, 0, 0)),
        ],
        out_specs=pl.BlockSpec((1, _C, _M), lambda b: (b, 0, 0)),
        out_shape=jax.ShapeDtypeStruct((_B, _C, _M), jnp.float32),
        compiler_params=pltpu.CompilerParams(
            dimension_semantics=("arbitrary",)),
    )(wv, xg)


def kernel(x, bin_tokens, Wq, Wk, Wv):
    aps8, bp8, xt = _stats_call(x, bin_tokens, Wq, Wk)
    rank3, cumks3 = _rank_call(aps8, bp8)
    rank = rank3.reshape(_B, _N)
    cumks = cumks3.reshape(_B, 128)
    idx = _select_call(rank, cumks)
    xg = _gather_call(idx, xt)
    out = _v_call(Wv, xg)
    return out, idx
